# baseline (speedup 1.0000x reference)
import functools

import jax
import jax.numpy as jnp
from jax import lax
from jax.experimental import pallas as pl
from jax.experimental.pallas import tpu as pltpu
from jax.experimental.pallas import tpu_sc as plsc

N = 16384
D = 32
C = 128
B = 4096
NB = N // B
MIN_COUNT = 10
_HI = jax.lax.Precision.HIGHEST

NSUB = 16
CH = N // NSUB
NG = CH // 16

_mesh = plsc.VectorSubcoreMesh(core_axis_name="c", subcore_axis_name="s")


@functools.partial(
    pl.kernel,
    mesh=_mesh,
    compiler_params=pltpu.CompilerParams(needs_layout_passes=False),
    out_type=(
        jax.ShapeDtypeStruct((N,), jnp.int32),
        jax.ShapeDtypeStruct((N,), jnp.int32),
        jax.ShapeDtypeStruct((C,), jnp.int32),
    ),
    scratch_types=(
        pltpu.VMEM((CH,), jnp.int32),
        pltpu.VMEM((CH,), jnp.int32),
        pltpu.VMEM((C,), jnp.int32),
        pltpu.VMEM((NSUB, C), jnp.int32),
        pltpu.VMEM_SHARED((NSUB, C), jnp.int32),
        pltpu.VMEM((C,), jnp.int32),
        pltpu.VMEM((C,), jnp.int32),
        pltpu.VMEM((16,), jnp.int32),
        pltpu.VMEM((16,), jnp.int32),
        pltpu.VMEM((16,), jnp.int32),
        pltpu.VMEM((CH,), jnp.int32),
        pltpu.VMEM((CH,), jnp.int32),
    ),
)
def _sc_group(t_hbm, ts_hbm, tq_hbm, counts_hbm,
              t_v, rank_v, hist, allh, shared, off_v, tot_v,
              tmp_a, tmp_b, tmp_c, ts_v, tq_v):
    cid = lax.axis_index("c")
    sid = lax.axis_index("s")
    base = sid * CH
    iota16 = lax.broadcasted_iota(jnp.int32, (16,), 0)
    zeros16 = jnp.zeros((16,), jnp.int32)

    pltpu.sync_copy(t_hbm.at[pl.ds(base, CH)], t_v)
    for k in range(C // 16):
        hist[pl.ds(k * 16, 16)] = zeros16

    def grp(g, _):
        tt = t_v[pl.ds(g * 16, 16)]
        cnt = plsc.load_gather(hist, [tt])
        sk, sv = plsc.sort_key_val(tt, iota16)
        tmp_a[...] = sk
        skprev = plsc.load_gather(tmp_a, [jnp.maximum(iota16 - 1, 0)])
        isnew = (iota16 == 0) | (sk != skprev)
        runstart = plsc.cummax(jnp.where(isnew, iota16, 0))
        runpos = iota16 - runstart
        plsc.store_scatter(tmp_b, [sv], runpos)
        delta = tmp_b[...]
        rank_v[pl.ds(g * 16, 16)] = cnt + delta
        tmp_c[...] = cnt
        cnt_sorted = plsc.load_gather(tmp_c, [sv])
        tmp_a[...] = jnp.where(isnew, 1, 0)
        nxt = plsc.load_gather(tmp_a, [jnp.minimum(iota16 + 1, 15)])
        lastrun = (iota16 == 15) | (nxt == 1)
        plsc.store_scatter(hist, [sk], cnt_sorted + runpos + 1, mask=lastrun)
        return 0

    lax.fori_loop(0, NG, grp, 0)

    pltpu.sync_copy(hist, shared.at[sid])
    plsc.subcore_barrier()
    pltpu.sync_copy(shared, allh)

    for k in range(C // 16):
        def red(w, carry):
            off, tot = carry
            v = allh[w, pl.ds(k * 16, 16)]
            return off + v * jnp.where(w < sid, 1, 0), tot + v

        off, tot = lax.fori_loop(0, NSUB, red, (zeros16, zeros16))
        off_v[pl.ds(k * 16, 16)] = off
        tot_v[pl.ds(k * 16, 16)] = tot

    @pl.when((cid == 0) & (sid == 0))
    def _():
        pltpu.sync_copy(tot_v, counts_hbm)

    def grp2(g, _):
        tt = t_v[pl.ds(g * 16, 16)]
        r = rank_v[pl.ds(g * 16, 16)] + plsc.load_gather(off_v, [tt])
        ctot = plsc.load_gather(tot_v, [tt])
        nsv = lax.shift_right_arithmetic(ctot, 1)
        vld = ctot >= MIN_COUNT
        ts_v[pl.ds(g * 16, 16)] = jnp.where(vld & (r < nsv), tt, C)
        tq_v[pl.ds(g * 16, 16)] = jnp.where(vld & (r >= nsv), tt, C)
        return 0

    lax.fori_loop(0, NG, grp2, 0)

    @pl.when(cid == 0)
    def _():
        pltpu.sync_copy(ts_v, ts_hbm.at[pl.ds(base, CH)])
        pltpu.sync_copy(tq_v, tq_hbm.at[pl.ds(base, CH)])


def _tc_body(x_ref, ts_ref, tq_ref, cnt_ref, out_ref):
    iota_c = jax.lax.broadcasted_iota(jnp.int32, (1, C), 1)
    counts = cnt_ref[...].astype(jnp.float32)
    ns = jnp.floor(counts * 0.5)
    valid = counts >= float(MIN_COUNT)

    iota_cb = jax.lax.broadcasted_iota(jnp.int32, (C, B), 0)

    def ph2(i, acc):
        oh_sup = (iota_cb == ts_ref[i][None, :]).astype(jnp.bfloat16)
        x_blk = x_ref[i]
        xh = x_blk.astype(jnp.bfloat16)
        r = x_blk - xh.astype(jnp.float32)
        xm = r.astype(jnp.bfloat16)
        xl = (r - xm.astype(jnp.float32)).astype(jnp.bfloat16)
        dn = (((1,), (0,)), ((), ()))
        for xs in (xh, xm, xl):
            acc = acc + jax.lax.dot_general(
                oh_sup, xs, dn, preferred_element_type=jnp.float32)
        return acc

    psum = jnp.zeros((C, D), jnp.float32)
    for _i in range(NB):
        psum = ph2(_i, psum)

    inv_n = 1.0 / jnp.maximum(ns, 1.0)
    inv2 = 2.0 * inv_n
    neg_inf = jnp.float32(-jnp.inf)
    sn2 = jnp.where(valid,
                    jnp.sum(psum * psum, axis=1).reshape(1, C) * inv_n * inv_n,
                    jnp.float32(jnp.inf))
    enc = (float(C - 1) - iota_c).astype(jnp.float32)

    def ph3(i, carry):
        loss_vec, acc_vec, q_vec = carry
        x_blk = x_ref[i]
        oh_q = tq_ref[i][:, None] == iota_c
        g = jax.lax.dot_general(x_blk, psum, (((1,), (1,)), ((), ())),
                                precision=_HI)
        logits = g * inv2 - sn2
        m = jnp.max(logits, axis=1, keepdims=True)
        ssum = jnp.sum(jnp.exp(logits - m), axis=1, keepdims=True)
        logit_t = jnp.sum(jnp.where(oh_q, logits, 0.0), axis=1, keepdims=True)
        logp_t = logit_t - m - jnp.log(ssum)

        t_enc = jnp.max(jnp.where(oh_q, enc, -1.0), axis=1, keepdims=True)
        q_b = t_enc >= 0.0
        pred_enc = jnp.max(jnp.where(logits == m, enc, -1.0), axis=1,
                           keepdims=True)
        loss_vec += jnp.sum(jnp.where(q_b, -logp_t, 0.0))
        q_vec += jnp.sum(jnp.where(q_b, 1.0, 0.0))
        acc_vec += jnp.sum(jnp.where((pred_enc == t_enc) & q_b, 1.0, 0.0))
        return loss_vec, acc_vec, q_vec

    carry = (jnp.float32(0), jnp.float32(0), jnp.float32(0))
    for _i in range(NB):
        carry = ph3(_i, carry)
    loss_sum, acc_sum, qcnt = carry
    iota2 = jax.lax.broadcasted_iota(jnp.int32, (1, 2), 1)
    out_ref[...] = jnp.where(iota2 == 0, loss_sum / qcnt, acc_sum / qcnt)


@jax.jit
def kernel(input, target):
    t = target.astype(jnp.int32)
    ts, tq, cnts = _sc_group(t)
    out = pl.pallas_call(
        _tc_body,
        out_shape=jax.ShapeDtypeStruct((1, 2), jnp.float32),
    )(input.reshape(NB, B, D), ts.reshape(NB, B), tq.reshape(NB, B),
      cnts.reshape(1, C))
    return out[0, 0], out[0, 1]


# speedup vs baseline: 1.0295x; 1.0295x over previous
import functools

import jax
import jax.numpy as jnp
from jax import lax
from jax.experimental import pallas as pl
from jax.experimental.pallas import tpu as pltpu
from jax.experimental.pallas import tpu_sc as plsc

N = 16384
D = 32
C = 128
B = 2048
NB = N // B
MIN_COUNT = 10
_HI = jax.lax.Precision.HIGHEST

NSUB = 16
CH = N // NSUB
NG = CH // 16

_mesh = plsc.VectorSubcoreMesh(core_axis_name="c", subcore_axis_name="s")


@functools.partial(
    pl.kernel,
    mesh=_mesh,
    compiler_params=pltpu.CompilerParams(needs_layout_passes=False),
    out_type=(
        jax.ShapeDtypeStruct((N,), jnp.int32),
        jax.ShapeDtypeStruct((N,), jnp.int32),
        jax.ShapeDtypeStruct((C,), jnp.int32),
    ),
    scratch_types=(
        pltpu.VMEM((CH,), jnp.int32),
        pltpu.VMEM((CH,), jnp.int32),
        pltpu.VMEM((C,), jnp.int32),
        pltpu.VMEM((NSUB, C), jnp.int32),
        pltpu.VMEM_SHARED((NSUB, C), jnp.int32),
        pltpu.VMEM((C,), jnp.int32),
        pltpu.VMEM((C,), jnp.int32),
        pltpu.VMEM((16,), jnp.int32),
        pltpu.VMEM((16,), jnp.int32),
        pltpu.VMEM((16,), jnp.int32),
        pltpu.VMEM((CH,), jnp.int32),
        pltpu.VMEM((CH,), jnp.int32),
    ),
)
def _sc_group(t_hbm, ts_hbm, tq_hbm, counts_hbm,
              t_v, rank_v, hist, allh, shared, off_v, tot_v,
              tmp_a, tmp_b, tmp_c, ts_v, tq_v):
    cid = lax.axis_index("c")
    sid = lax.axis_index("s")
    base = sid * CH
    iota16 = lax.broadcasted_iota(jnp.int32, (16,), 0)
    zeros16 = jnp.zeros((16,), jnp.int32)

    pltpu.sync_copy(t_hbm.at[pl.ds(base, CH)], t_v)
    for k in range(C // 16):
        hist[pl.ds(k * 16, 16)] = zeros16

    def grp(g, _):
        tt = t_v[pl.ds(g * 16, 16)]
        cnt = plsc.load_gather(hist, [tt])
        sk, sv = plsc.sort_key_val(tt, iota16)
        tmp_a[...] = sk
        skprev = plsc.load_gather(tmp_a, [jnp.maximum(iota16 - 1, 0)])
        isnew = (iota16 == 0) | (sk != skprev)
        runstart = plsc.cummax(jnp.where(isnew, iota16, 0))
        runpos = iota16 - runstart
        plsc.store_scatter(tmp_b, [sv], runpos)
        delta = tmp_b[...]
        rank_v[pl.ds(g * 16, 16)] = cnt + delta
        tmp_c[...] = cnt
        cnt_sorted = plsc.load_gather(tmp_c, [sv])
        tmp_a[...] = jnp.where(isnew, 1, 0)
        nxt = plsc.load_gather(tmp_a, [jnp.minimum(iota16 + 1, 15)])
        lastrun = (iota16 == 15) | (nxt == 1)
        plsc.store_scatter(hist, [sk], cnt_sorted + runpos + 1, mask=lastrun)
        return 0

    lax.fori_loop(0, NG, grp, 0)

    pltpu.sync_copy(hist, shared.at[sid])
    plsc.subcore_barrier()
    pltpu.sync_copy(shared, allh)

    for k in range(C // 16):
        def red(w, carry):
            off, tot = carry
            v = allh[w, pl.ds(k * 16, 16)]
            return off + v * jnp.where(w < sid, 1, 0), tot + v

        off, tot = lax.fori_loop(0, NSUB, red, (zeros16, zeros16))
        off_v[pl.ds(k * 16, 16)] = off
        tot_v[pl.ds(k * 16, 16)] = tot

    @pl.when((cid == 0) & (sid == 0))
    def _():
        pltpu.sync_copy(tot_v, counts_hbm)

    def grp2(g, _):
        tt = t_v[pl.ds(g * 16, 16)]
        r = rank_v[pl.ds(g * 16, 16)] + plsc.load_gather(off_v, [tt])
        ctot = plsc.load_gather(tot_v, [tt])
        nsv = lax.shift_right_arithmetic(ctot, 1)
        vld = ctot >= MIN_COUNT
        ts_v[pl.ds(g * 16, 16)] = jnp.where(vld & (r < nsv), tt, C)
        tq_v[pl.ds(g * 16, 16)] = jnp.where(vld & (r >= nsv), tt, C)
        return 0

    lax.fori_loop(0, NG, grp2, 0)

    @pl.when(cid == 0)
    def _():
        pltpu.sync_copy(ts_v, ts_hbm.at[pl.ds(base, CH)])
        pltpu.sync_copy(tq_v, tq_hbm.at[pl.ds(base, CH)])


def _tc_body(x_ref, ts_ref, tq_ref, cnt_ref, out_ref):
    iota_c = jax.lax.broadcasted_iota(jnp.int32, (1, C), 1)
    counts = cnt_ref[...].astype(jnp.float32)
    ns = jnp.floor(counts * 0.5)
    valid = counts >= float(MIN_COUNT)

    iota_cb = jax.lax.broadcasted_iota(jnp.int32, (C, B), 0)

    def ph2(i, acc):
        oh_sup = (iota_cb == ts_ref[i][None, :]).astype(jnp.bfloat16)
        x_blk = x_ref[i]
        xh = x_blk.astype(jnp.bfloat16)
        r = x_blk - xh.astype(jnp.float32)
        xm = r.astype(jnp.bfloat16)
        xl = (r - xm.astype(jnp.float32)).astype(jnp.bfloat16)
        dn = (((1,), (0,)), ((), ()))
        for xs in (xh, xm, xl):
            acc = acc + jax.lax.dot_general(
                oh_sup, xs, dn, preferred_element_type=jnp.float32)
        return acc

    psum = jnp.zeros((C, D), jnp.float32)
    for _i in range(NB):
        psum = ph2(_i, psum)

    inv_n = 1.0 / jnp.maximum(ns, 1.0)
    inv2 = 2.0 * inv_n
    neg_inf = jnp.float32(-jnp.inf)
    sn2 = jnp.where(valid,
                    jnp.sum(psum * psum, axis=1).reshape(1, C) * inv_n * inv_n,
                    jnp.float32(jnp.inf))
    enc = (float(C - 1) - iota_c).astype(jnp.float32)

    def ph3(i, carry):
        loss_vec, acc_vec, q_vec = carry
        x_blk = x_ref[i]
        oh_q = tq_ref[i][:, None] == iota_c
        g = jax.lax.dot_general(x_blk, psum, (((1,), (1,)), ((), ())),
                                precision=_HI)
        logits = g * inv2 - sn2
        m = jnp.max(logits, axis=1, keepdims=True)
        eb = jnp.exp((logits - m).astype(jnp.bfloat16))
        ssum = jnp.sum(eb, axis=1, keepdims=True, dtype=jnp.float32)
        logit_t = jnp.sum(jnp.where(oh_q, logits, 0.0), axis=1, keepdims=True)
        logp_t = logit_t - m - jnp.log(ssum)

        t_enc = jnp.max(jnp.where(oh_q, enc, -1.0), axis=1, keepdims=True)
        q_b = t_enc >= 0.0
        pred_enc = jnp.max(jnp.where(logits == m, enc, -1.0), axis=1,
                           keepdims=True)
        loss_vec += jnp.sum(jnp.where(q_b, -logp_t, 0.0))
        q_vec += jnp.sum(jnp.where(q_b, 1.0, 0.0))
        acc_vec += jnp.sum(jnp.where((pred_enc == t_enc) & q_b, 1.0, 0.0))
        return loss_vec, acc_vec, q_vec

    carry = (jnp.float32(0), jnp.float32(0), jnp.float32(0))
    for _i in range(NB):
        carry = ph3(_i, carry)
    loss_sum, acc_sum, qcnt = carry
    iota2 = jax.lax.broadcasted_iota(jnp.int32, (1, 2), 1)
    out_ref[...] = jnp.where(iota2 == 0, loss_sum / qcnt, acc_sum / qcnt)


@jax.jit
def kernel(input, target):
    t = target.astype(jnp.int32)
    ts, tq, cnts = _sc_group(t)
    out = pl.pallas_call(
        _tc_body,
        out_shape=jax.ShapeDtypeStruct((1, 2), jnp.float32),
    )(input.reshape(NB, B, D), ts.reshape(NB, B), tq.reshape(NB, B),
      cnts.reshape(1, C))
    return out[0, 0], out[0, 1]


# speedup vs baseline: 1.0326x; 1.0030x over previous
import functools

import jax
import jax.numpy as jnp
from jax import lax
from jax.experimental import pallas as pl
from jax.experimental.pallas import tpu as pltpu
from jax.experimental.pallas import tpu_sc as plsc

N = 16384
D = 32
C = 128
B = 2048
NB = N // B
MIN_COUNT = 10
_HI = jax.lax.Precision.HIGHEST

NSUB = 16
CH = N // NSUB
NG = CH // 16

_mesh = plsc.VectorSubcoreMesh(core_axis_name="c", subcore_axis_name="s")


@functools.partial(
    pl.kernel,
    mesh=_mesh,
    compiler_params=pltpu.CompilerParams(needs_layout_passes=False),
    out_type=(
        jax.ShapeDtypeStruct((N,), jnp.int32),
        jax.ShapeDtypeStruct((N,), jnp.int32),
        jax.ShapeDtypeStruct((C,), jnp.int32),
    ),
    scratch_types=(
        pltpu.VMEM((CH,), jnp.int32),
        pltpu.VMEM((CH,), jnp.int32),
        pltpu.VMEM((C,), jnp.int32),
        pltpu.VMEM((NSUB, C), jnp.int32),
        pltpu.VMEM_SHARED((NSUB, C), jnp.int32),
        pltpu.VMEM((C,), jnp.int32),
        pltpu.VMEM((C,), jnp.int32),
        pltpu.VMEM((16,), jnp.int32),
        pltpu.VMEM((16,), jnp.int32),
        pltpu.VMEM((16,), jnp.int32),
        pltpu.VMEM((CH,), jnp.int32),
        pltpu.VMEM((CH,), jnp.int32),
    ),
)
def _sc_group(t_hbm, ts_hbm, tq_hbm, counts_hbm,
              t_v, rank_v, hist, allh, shared, off_v, tot_v,
              tmp_a, tmp_b, tmp_c, ts_v, tq_v):
    cid = lax.axis_index("c")
    sid = lax.axis_index("s")
    base = sid * CH
    iota16 = lax.broadcasted_iota(jnp.int32, (16,), 0)
    zeros16 = jnp.zeros((16,), jnp.int32)

    pltpu.sync_copy(t_hbm.at[pl.ds(base, CH)], t_v)
    for k in range(C // 16):
        hist[pl.ds(k * 16, 16)] = zeros16

    def grp(g, _):
        tt = t_v[pl.ds(g * 16, 16)]
        cnt = plsc.load_gather(hist, [tt])
        sk, sv = plsc.sort_key_val(tt, iota16)
        tmp_a[...] = sk
        skprev = plsc.load_gather(tmp_a, [jnp.maximum(iota16 - 1, 0)])
        isnew = (iota16 == 0) | (sk != skprev)
        runstart = plsc.cummax(jnp.where(isnew, iota16, 0))
        runpos = iota16 - runstart
        plsc.store_scatter(tmp_b, [sv], runpos)
        delta = tmp_b[...]
        rank_v[pl.ds(g * 16, 16)] = cnt + delta
        tmp_c[...] = cnt
        cnt_sorted = plsc.load_gather(tmp_c, [sv])
        tmp_a[...] = jnp.where(isnew, 1, 0)
        nxt = plsc.load_gather(tmp_a, [jnp.minimum(iota16 + 1, 15)])
        lastrun = (iota16 == 15) | (nxt == 1)
        plsc.store_scatter(hist, [sk], cnt_sorted + runpos + 1, mask=lastrun)
        return 0

    lax.fori_loop(0, NG, grp, 0)

    pltpu.sync_copy(hist, shared.at[sid])
    plsc.subcore_barrier()
    pltpu.sync_copy(shared, allh)

    for k in range(C // 16):
        def red(w, carry):
            off, tot = carry
            v = allh[w, pl.ds(k * 16, 16)]
            return off + v * jnp.where(w < sid, 1, 0), tot + v

        off, tot = lax.fori_loop(0, NSUB, red, (zeros16, zeros16))
        off_v[pl.ds(k * 16, 16)] = off
        tot_v[pl.ds(k * 16, 16)] = tot

    @pl.when((cid == 0) & (sid == 0))
    def _():
        pltpu.sync_copy(tot_v, counts_hbm)

    def grp2(g, _):
        tt = t_v[pl.ds(g * 16, 16)]
        r = rank_v[pl.ds(g * 16, 16)] + plsc.load_gather(off_v, [tt])
        ctot = plsc.load_gather(tot_v, [tt])
        nsv = lax.shift_right_arithmetic(ctot, 1)
        vld = ctot >= MIN_COUNT
        ts_v[pl.ds(g * 16, 16)] = jnp.where(vld & (r < nsv), tt, C)
        tq_v[pl.ds(g * 16, 16)] = jnp.where(vld & (r >= nsv), tt, C)
        return 0

    lax.fori_loop(0, NG, grp2, 0)

    @pl.when(cid == 0)
    def _():
        pltpu.sync_copy(ts_v, ts_hbm.at[pl.ds(base, CH)])
        pltpu.sync_copy(tq_v, tq_hbm.at[pl.ds(base, CH)])


def _tc_body(x_ref, ts_ref, tq_ref, cnt_ref, out_ref):
    iota_c = jax.lax.broadcasted_iota(jnp.int32, (1, C), 1)
    counts = cnt_ref[...].astype(jnp.float32)
    ns = jnp.floor(counts * 0.5)
    valid = counts >= float(MIN_COUNT)

    iota_cb = jax.lax.broadcasted_iota(jnp.int32, (C, B), 0)

    def ph2(i, acc):
        oh_sup = (iota_cb == ts_ref[i][None, :]).astype(jnp.bfloat16)
        x_blk = x_ref[i]
        xh = x_blk.astype(jnp.bfloat16)
        r = x_blk - xh.astype(jnp.float32)
        xm = r.astype(jnp.bfloat16)
        xl = (r - xm.astype(jnp.float32)).astype(jnp.bfloat16)
        dn = (((1,), (0,)), ((), ()))
        for xs in (xh, xm, xl):
            acc = acc + jax.lax.dot_general(
                oh_sup, xs, dn, preferred_element_type=jnp.float32)
        return acc

    psum = jnp.zeros((C, D), jnp.float32)
    for _i in range(NB):
        psum = ph2(_i, psum)

    inv_n = 1.0 / jnp.maximum(ns, 1.0)
    inv2 = 2.0 * inv_n
    neg_inf = jnp.float32(-jnp.inf)
    sn2 = jnp.where(valid,
                    jnp.sum(psum * psum, axis=1).reshape(1, C) * inv_n * inv_n,
                    jnp.float32(jnp.inf))
    enc = (float(C - 1) - iota_c).astype(jnp.float32)

    def ph3(i, carry):
        loss_vec, acc_vec, q_vec = carry
        x_blk = x_ref[i]
        oh_q = tq_ref[i][:, None] == iota_c
        g = jax.lax.dot_general(x_blk, psum, (((1,), (1,)), ((), ())),
                                precision=_HI)
        logits = g * inv2 - sn2
        m = jnp.max(logits, axis=1, keepdims=True)
        ssum = jnp.sum(jnp.exp(logits - m), axis=1, keepdims=True)
        logit_t = jnp.sum(jnp.where(oh_q, logits, 0.0), axis=1, keepdims=True)
        logp_t = logit_t - m - jnp.log(ssum)

        t_enc = jnp.max(jnp.where(oh_q, enc, -1.0), axis=1, keepdims=True)
        q_b = t_enc >= 0.0
        pred_enc = jnp.max(jnp.where(logits == m, enc, -1.0), axis=1,
                           keepdims=True)
        loss_vec += jnp.sum(jnp.where(q_b, -logp_t, 0.0))
        q_vec += jnp.sum(jnp.where(q_b, 1.0, 0.0))
        acc_vec += jnp.sum(jnp.where((pred_enc == t_enc) & q_b, 1.0, 0.0))
        return loss_vec, acc_vec, q_vec

    carry = (jnp.float32(0), jnp.float32(0), jnp.float32(0))
    for _i in range(NB):
        carry = ph3(_i, carry)
    loss_sum, acc_sum, qcnt = carry
    iota2 = jax.lax.broadcasted_iota(jnp.int32, (1, 2), 1)
    out_ref[...] = jnp.where(iota2 == 0, loss_sum / qcnt, acc_sum / qcnt)


@jax.jit
def kernel(input, target):
    t = target.astype(jnp.int32)
    ts, tq, cnts = _sc_group(t)
    out = pl.pallas_call(
        _tc_body,
        out_shape=jax.ShapeDtypeStruct((1, 2), jnp.float32),
    )(input.reshape(NB, B, D), ts.reshape(NB, B), tq.reshape(NB, B),
      cnts.reshape(1, C))
    return out[0, 0], out[0, 1]


# speedup vs baseline: 1.0329x; 1.0003x over previous
import functools

import jax
import jax.numpy as jnp
from jax import lax
from jax.experimental import pallas as pl
from jax.experimental.pallas import tpu as pltpu
from jax.experimental.pallas import tpu_sc as plsc

N = 16384
D = 32
C = 128
B = 2048
NB = N // B
MIN_COUNT = 10
_HI = jax.lax.Precision.HIGHEST

NSUB = 16
CH = N // NSUB
NG = CH // 16

_mesh = plsc.VectorSubcoreMesh(core_axis_name="c", subcore_axis_name="s")


@functools.partial(
    pl.kernel,
    mesh=_mesh,
    compiler_params=pltpu.CompilerParams(needs_layout_passes=False),
    out_type=(
        jax.ShapeDtypeStruct((N,), jnp.int32),
        jax.ShapeDtypeStruct((N,), jnp.int32),
        jax.ShapeDtypeStruct((C,), jnp.int32),
    ),
    scratch_types=(
        pltpu.VMEM((CH,), jnp.int32),
        pltpu.VMEM((CH,), jnp.int32),
        pltpu.VMEM((C,), jnp.int32),
        pltpu.VMEM((NSUB, C), jnp.int32),
        pltpu.VMEM_SHARED((NSUB, C), jnp.int32),
        pltpu.VMEM((C,), jnp.int32),
        pltpu.VMEM((C,), jnp.int32),
        pltpu.VMEM((16,), jnp.int32),
        pltpu.VMEM((16,), jnp.int32),
        pltpu.VMEM((16,), jnp.int32),
        pltpu.VMEM((CH,), jnp.int32),
        pltpu.VMEM((CH,), jnp.int32),
    ),
)
def _sc_group(t_hbm, ts_hbm, tq_hbm, counts_hbm,
              t_v, rank_v, hist, allh, shared, off_v, tot_v,
              tmp_a, tmp_b, tmp_c, ts_v, tq_v):
    cid = lax.axis_index("c")
    sid = lax.axis_index("s")
    base = sid * CH
    iota16 = lax.broadcasted_iota(jnp.int32, (16,), 0)
    zeros16 = jnp.zeros((16,), jnp.int32)

    pltpu.sync_copy(t_hbm.at[pl.ds(base, CH)], t_v)
    for k in range(C // 16):
        hist[pl.ds(k * 16, 16)] = zeros16

    def grp(g, _):
        tt = t_v[pl.ds(g * 16, 16)]
        cnt = plsc.load_gather(hist, [tt])
        sk, sv = plsc.sort_key_val(tt, iota16)
        tmp_a[...] = sk
        skprev = plsc.load_gather(tmp_a, [jnp.maximum(iota16 - 1, 0)])
        isnew = (iota16 == 0) | (sk != skprev)
        runstart = plsc.cummax(jnp.where(isnew, iota16, 0))
        runpos = iota16 - runstart
        plsc.store_scatter(tmp_b, [sv], runpos)
        delta = tmp_b[...]
        rank_v[pl.ds(g * 16, 16)] = cnt + delta
        tmp_c[...] = cnt
        cnt_sorted = plsc.load_gather(tmp_c, [sv])
        tmp_a[...] = jnp.where(isnew, 1, 0)
        nxt = plsc.load_gather(tmp_a, [jnp.minimum(iota16 + 1, 15)])
        lastrun = (iota16 == 15) | (nxt == 1)
        plsc.store_scatter(hist, [sk], cnt_sorted + runpos + 1, mask=lastrun)
        return 0

    lax.fori_loop(0, NG, grp, 0)

    pltpu.sync_copy(hist, shared.at[sid])
    plsc.subcore_barrier()
    pltpu.sync_copy(shared, allh)

    for k in range(C // 16):
        def red(w, carry):
            off, tot = carry
            v = allh[w, pl.ds(k * 16, 16)]
            return off + v * jnp.where(w < sid, 1, 0), tot + v

        off, tot = lax.fori_loop(0, NSUB, red, (zeros16, zeros16))
        off_v[pl.ds(k * 16, 16)] = off
        tot_v[pl.ds(k * 16, 16)] = tot

    @pl.when((cid == 0) & (sid == 0))
    def _():
        pltpu.sync_copy(tot_v, counts_hbm)

    def grp2(g, _):
        tt = t_v[pl.ds(g * 16, 16)]
        r = rank_v[pl.ds(g * 16, 16)] + plsc.load_gather(off_v, [tt])
        ctot = plsc.load_gather(tot_v, [tt])
        nsv = lax.shift_right_arithmetic(ctot, 1)
        vld = ctot >= MIN_COUNT
        ts_v[pl.ds(g * 16, 16)] = jnp.where(vld & (r < nsv), tt, C)
        tq_v[pl.ds(g * 16, 16)] = jnp.where(vld & (r >= nsv), tt, C)
        return 0

    lax.fori_loop(0, NG, grp2, 0)

    @pl.when(cid == 0)
    def _():
        pltpu.sync_copy(ts_v, ts_hbm.at[pl.ds(base, CH)])
        pltpu.sync_copy(tq_v, tq_hbm.at[pl.ds(base, CH)])


def _tc_body(x_ref, ts_ref, tq_ref, cnt_ref, out_ref):
    iota_c = jax.lax.broadcasted_iota(jnp.int32, (1, C), 1)
    counts = cnt_ref[...].astype(jnp.float32)
    ns = jnp.floor(counts * 0.5)
    valid = counts >= float(MIN_COUNT)

    iota_cb = jax.lax.broadcasted_iota(jnp.int32, (C, B), 0)

    def ph2(i, acc):
        oh_sup = (iota_cb == ts_ref[i][None, :]).astype(jnp.bfloat16)
        x_blk = x_ref[i]
        xh = x_blk.astype(jnp.bfloat16)
        r = x_blk - xh.astype(jnp.float32)
        xm = r.astype(jnp.bfloat16)
        xl = (r - xm.astype(jnp.float32)).astype(jnp.bfloat16)
        dn = (((1,), (0,)), ((), ()))
        for xs in (xh, xm, xl):
            acc = acc + jax.lax.dot_general(
                oh_sup, xs, dn, preferred_element_type=jnp.float32)
        return acc

    psum = jnp.zeros((C, D), jnp.float32)
    for _i in range(NB):
        psum = ph2(_i, psum)

    inv_n = 1.0 / jnp.maximum(ns, 1.0)
    inv2 = 2.0 * inv_n
    sn2 = jnp.where(valid,
                    jnp.sum(psum * psum, axis=1).reshape(1, C) * inv_n * inv_n,
                    jnp.float32(jnp.inf))
    enc = (float(C - 1) - iota_c).astype(jnp.float32)

    def ph3(i, carry):
        loss_sum, acc_sum, qcnt = carry
        x_blk = x_ref[i]
        oh_q = tq_ref[i][:, None] == iota_c
        g = jax.lax.dot_general(x_blk, psum, (((1,), (1,)), ((), ())),
                                precision=_HI)
        logits = g * inv2 - sn2
        m = jnp.max(logits, axis=1, keepdims=True)
        ssum = jnp.sum(jnp.exp(logits - m), axis=1, keepdims=True)
        logit_t = jnp.sum(jnp.where(oh_q, logits, 0.0), axis=1, keepdims=True)
        logp_t = logit_t - m - jnp.log(ssum)

        t_enc = jnp.max(jnp.where(oh_q, enc, -1.0), axis=1, keepdims=True)
        q_b = t_enc >= 0.0
        pred_enc = jnp.max(jnp.where(logits == m, enc, -1.0), axis=1,
                           keepdims=True)
        loss_sum += jnp.sum(jnp.where(q_b, -logp_t, 0.0))
        qcnt += jnp.sum(jnp.where(q_b, 1.0, 0.0))
        acc_sum += jnp.sum(jnp.where((pred_enc == t_enc) & q_b, 1.0, 0.0))
        return loss_sum, acc_sum, qcnt

    carry = (jnp.float32(0), jnp.float32(0), jnp.float32(0))
    for _i in range(NB):
        carry = ph3(_i, carry)
    loss_sum, acc_sum, qcnt = carry
    iota2 = jax.lax.broadcasted_iota(jnp.int32, (1, 2), 1)
    out_ref[...] = jnp.where(iota2 == 0, loss_sum / qcnt, acc_sum / qcnt)


@jax.jit
def kernel(input, target):
    t = target.astype(jnp.int32)
    ts, tq, cnts = _sc_group(t)
    out = pl.pallas_call(
        _tc_body,
        out_shape=jax.ShapeDtypeStruct((1, 2), jnp.float32),
    )(input.reshape(NB, B, D), ts.reshape(NB, B), tq.reshape(NB, B),
      cnts.reshape(1, C))
    return out[0, 0], out[0, 1]
